# Initial kernel scaffold; baseline (speedup 1.0000x reference)
import jax
import jax.numpy as jnp
from jax.experimental import pallas as pl


def kernel(x, pos, batch, W0, b0, g0, bt0, W1, b1, g1, bt1, W2, b2, g2, bt2):
    raise NotImplementedError("write your pallas kernel here")


# speedup vs baseline: 1.6404x; 1.6404x over previous
import jax
import jax.numpy as jnp
from jax import lax
from jax.experimental import pallas as pl

_B, _P, _F = 8, 2048, 3
_M, _K = 1024, 64
_R2 = 0.2 * 0.2
_C0, _C1, _C2 = 64, 64, 128
_E = _B * _K * _M
_T = 4096
_TM = 256
_EPS = 1e-5


def _select_kernel(pos_ref, idx_ref, cen_ref, nbr_ref, val_ref):
    posv = pos_ref[0]
    iota_p = lax.broadcasted_iota(jnp.int32, (1, _P), 1)
    p0 = posv[0:1, :]
    mind0 = jnp.sum((posv - p0) ** 2, axis=1)[None, :]
    cen_ref[0, 0:1, :] = p0
    idx_ref[0, 0:1, :] = jnp.zeros((1, 1), jnp.int32)

    def fps_body(i, mind):
        mx = jnp.max(mind)
        nxt = jnp.min(jnp.where(mind == mx, iota_p, _P)).astype(jnp.int32)
        prow = pos_ref[0, pl.ds(nxt, 1), :]
        d = jnp.sum((posv - prow) ** 2, axis=1)[None, :]
        cen_ref[0, pl.ds(i, 1), :] = prow
        idx_ref[0, pl.ds(i, 1), :] = nxt.reshape(1, 1)
        return jnp.minimum(mind, d)

    lax.fori_loop(1, _M, fps_body, mind0)
    cen = cen_ref[0]

    d2 = jnp.zeros((_M, _P), jnp.float32)
    for f in range(3):
        d2 = d2 + (cen[:, f][:, None] - posv[:, f][None, :]) ** 2
    d2 = jnp.where(d2 <= _R2, d2, jnp.inf)
    col = lax.broadcasted_iota(jnp.int32, (_M, _P), 1)

    def topk_body(k, d2w):
        mn = jnp.min(d2w, axis=1, keepdims=True)
        amn = jnp.min(jnp.where(d2w == mn, col, _P), axis=1)
        nbr_ref[0, pl.ds(k, 1), :] = amn[None, :].astype(jnp.int32)
        val_ref[0, pl.ds(k, 1), :] = (
            (mn[:, 0] < jnp.inf).astype(jnp.float32)[None, :])
        return jnp.where(col == amn[:, None], jnp.inf, d2w)

    lax.fori_loop(0, _K, topk_body, d2)


def _l0_kernel(nbr_ref, val_ref, g_ref, cen_ref, w_ref, b_ref,
               h_ref, s_ref, ss_ref, cnt_ref):
    b = pl.program_id(0)
    k = pl.program_id(1)

    @pl.when(jnp.logical_and(b == 0, k == 0))
    def _init():
        s_ref[...] = jnp.zeros_like(s_ref)
        ss_ref[...] = jnp.zeros_like(ss_ref)
        cnt_ref[...] = jnp.zeros_like(cnt_ref)

    nbr = nbr_ref[0, 0, :, 0]
    G = g_ref[0]
    cen = cen_ref[0]
    onehot = (nbr[:, None] == lax.broadcasted_iota(jnp.int32, (_M, _P), 1)
              ).astype(jnp.float32)
    h0 = jnp.dot(onehot, G, preferred_element_type=jnp.float32)
    h0 = h0 - jnp.concatenate([jnp.zeros((_M, 3), jnp.float32), cen], axis=1)
    z = jnp.maximum(
        jnp.dot(h0, w_ref[...], preferred_element_type=jnp.float32) + b_ref[...],
        0.0)
    mf = val_ref[0, 0]
    zm = z * mf
    s_ref[...] += jnp.sum(zm, axis=0, keepdims=True)
    ss_ref[...] += jnp.sum(z * zm, axis=0, keepdims=True)
    cnt_ref[...] += jnp.sum(mf).reshape(1, 1)
    h_ref[0, 0] = z


def _mid_kernel(h_ref, v_ref, s_ref, ss_ref, cnt_ref, g_ref, bt_ref,
                w_ref, b_ref, o_ref, s2_ref, ss2_ref):
    @pl.when(pl.program_id(0) == 0)
    def _init():
        s2_ref[...] = jnp.zeros_like(s2_ref)
        ss2_ref[...] = jnp.zeros_like(ss2_ref)

    cnt = jnp.maximum(cnt_ref[0, 0], 1.0)
    mean = s_ref[...] / cnt
    var = jnp.maximum(ss_ref[...] / cnt - mean * mean, 0.0)
    hn = (h_ref[...] - mean) * (g_ref[...] * lax.rsqrt(var + _EPS)) + bt_ref[...]
    z = jnp.maximum(
        jnp.dot(hn, w_ref[...], preferred_element_type=jnp.float32) + b_ref[...],
        0.0)
    mf = v_ref[...]
    zm = z * mf
    s2_ref[...] += jnp.sum(zm, axis=0, keepdims=True)
    ss2_ref[...] += jnp.sum(z * zm, axis=0, keepdims=True)
    o_ref[...] = z


def _out_kernel(h_ref, v_ref, s_ref, ss_ref, cnt_ref, g_ref, bt_ref, o_ref):
    cnt = jnp.maximum(cnt_ref[0, 0], 1.0)
    mean = s_ref[...] / cnt
    var = jnp.maximum(ss_ref[...] / cnt - mean * mean, 0.0)
    scale = (g_ref[...] * lax.rsqrt(var + _EPS)).reshape(1, 1, _C2)
    h = h_ref[0]
    hn = (h - mean.reshape(1, 1, _C2)) * scale + bt_ref[...].reshape(1, 1, _C2)
    hn = jnp.where(v_ref[0][:, :, None] > 0.0, hn, -jnp.inf)
    o_ref[0] = jnp.max(hn, axis=0)


def _mid_call(hflat, vflat, s, ss, cnt, g, bt, W, b, cout):
    n_t = _E // _T
    cin = hflat.shape[1]
    return pl.pallas_call(
        _mid_kernel,
        grid=(n_t,),
        in_specs=[
            pl.BlockSpec((_T, cin), lambda i: (i, 0)),
            pl.BlockSpec((_T, 1), lambda i: (i, 0)),
            pl.BlockSpec((1, cin), lambda i: (0, 0)),
            pl.BlockSpec((1, cin), lambda i: (0, 0)),
            pl.BlockSpec((1, 1), lambda i: (0, 0)),
            pl.BlockSpec((1, cin), lambda i: (0, 0)),
            pl.BlockSpec((1, cin), lambda i: (0, 0)),
            pl.BlockSpec((cin, cout), lambda i: (0, 0)),
            pl.BlockSpec((1, cout), lambda i: (0, 0)),
        ],
        out_specs=[
            pl.BlockSpec((_T, cout), lambda i: (i, 0)),
            pl.BlockSpec((1, cout), lambda i: (0, 0)),
            pl.BlockSpec((1, cout), lambda i: (0, 0)),
        ],
        out_shape=[
            jax.ShapeDtypeStruct((_E, cout), jnp.float32),
            jax.ShapeDtypeStruct((1, cout), jnp.float32),
            jax.ShapeDtypeStruct((1, cout), jnp.float32),
        ],
    )(hflat, vflat, s, ss, cnt, g.reshape(1, -1), bt.reshape(1, -1),
      W, b.reshape(1, -1))


@jax.jit
def kernel(x, pos, batch, W0, b0, g0, bt0, W1, b1, g1, bt1, W2, b2, g2, bt2):
    x3 = x.reshape(_B, _P, _F)
    pos3 = pos.reshape(_B, _P, 3)

    idx, cen, nbr, val = pl.pallas_call(
        _select_kernel,
        grid=(_B,),
        in_specs=[pl.BlockSpec((1, _P, 3), lambda b: (b, 0, 0))],
        out_specs=[
            pl.BlockSpec((1, _M, 1), lambda b: (b, 0, 0)),
            pl.BlockSpec((1, _M, 3), lambda b: (b, 0, 0)),
            pl.BlockSpec((1, _K, _M), lambda b: (b, 0, 0)),
            pl.BlockSpec((1, _K, _M), lambda b: (b, 0, 0)),
        ],
        out_shape=[
            jax.ShapeDtypeStruct((_B, _M, 1), jnp.int32),
            jax.ShapeDtypeStruct((_B, _M, 3), jnp.float32),
            jax.ShapeDtypeStruct((_B, _K, _M), jnp.int32),
            jax.ShapeDtypeStruct((_B, _K, _M), jnp.float32),
        ],
    )(pos3)

    G = jnp.concatenate([x3, pos3], axis=-1)
    h1, s0, ss0, cnt = pl.pallas_call(
        _l0_kernel,
        grid=(_B, _K),
        in_specs=[
            pl.BlockSpec((1, 1, _M, 1), lambda b, k: (b, k, 0, 0)),
            pl.BlockSpec((1, 1, _M, 1), lambda b, k: (b, k, 0, 0)),
            pl.BlockSpec((1, _P, 6), lambda b, k: (b, 0, 0)),
            pl.BlockSpec((1, _M, 3), lambda b, k: (b, 0, 0)),
            pl.BlockSpec((6, _C0), lambda b, k: (0, 0)),
            pl.BlockSpec((1, _C0), lambda b, k: (0, 0)),
        ],
        out_specs=[
            pl.BlockSpec((1, 1, _M, _C0), lambda b, k: (b, k, 0, 0)),
            pl.BlockSpec((1, _C0), lambda b, k: (0, 0)),
            pl.BlockSpec((1, _C0), lambda b, k: (0, 0)),
            pl.BlockSpec((1, 1), lambda b, k: (0, 0)),
        ],
        out_shape=[
            jax.ShapeDtypeStruct((_B, _K, _M, _C0), jnp.float32),
            jax.ShapeDtypeStruct((1, _C0), jnp.float32),
            jax.ShapeDtypeStruct((1, _C0), jnp.float32),
            jax.ShapeDtypeStruct((1, 1), jnp.float32),
        ],
    )(nbr.reshape(_B, _K, _M, 1), val.reshape(_B, _K, _M, 1), G, cen,
      W0, b0.reshape(1, -1))

    vflat = val.reshape(_E, 1)
    h2, s1, ss1 = _mid_call(h1.reshape(_E, _C0), vflat, s0, ss0, cnt,
                            g0, bt0, W1, b1, _C1)
    h3, s2, ss2 = _mid_call(h2, vflat, s1, ss1, cnt, g1, bt1, W2, b2, _C2)

    out = pl.pallas_call(
        _out_kernel,
        grid=(_B, _M // _TM),
        in_specs=[
            pl.BlockSpec((1, _K, _TM, _C2), lambda b, m: (b, 0, m, 0)),
            pl.BlockSpec((1, _K, _TM), lambda b, m: (b, 0, m)),
            pl.BlockSpec((1, _C2), lambda b, m: (0, 0)),
            pl.BlockSpec((1, _C2), lambda b, m: (0, 0)),
            pl.BlockSpec((1, 1), lambda b, m: (0, 0)),
            pl.BlockSpec((1, _C2), lambda b, m: (0, 0)),
            pl.BlockSpec((1, _C2), lambda b, m: (0, 0)),
        ],
        out_specs=[pl.BlockSpec((1, _TM, _C2), lambda b, m: (b, m, 0))],
        out_shape=[jax.ShapeDtypeStruct((_B, _M, _C2), jnp.float32)],
    )(h3.reshape(_B, _K, _M, _C2), val, s2, ss2, cnt,
      g2.reshape(1, -1), bt2.reshape(1, -1))[0]

    x_out = out.reshape(_B * _M, _C2)
    pos_out = cen.reshape(_B * _M, 3)
    batch_out = jnp.repeat(jnp.arange(_B), _M)
    idx_g = (idx[:, :, 0] + (jnp.arange(_B) * _P)[:, None]).reshape(-1)
    return (x_out, pos_out, batch_out, idx_g)
